# baseline (speedup 1.0000x reference)
import functools

import jax
import jax.numpy as jnp
from jax import lax
from jax.experimental import pallas as pl
from jax.experimental.pallas import tpu as pltpu
from jax.experimental.pallas import tpu_sc as plsc

D = 64
NC = 2
NS = 16
NW = NC * NS
CH = 128


@functools.partial(jax.jit, static_argnames=("b_per_w",))
def _sc_gather(idx_flat, table, b_per_w):
    nch = b_per_w // CH
    mesh = plsc.VectorSubcoreMesh(core_axis_name="c", subcore_axis_name="s")

    @functools.partial(
        pl.kernel,
        out_type=jax.ShapeDtypeStruct((idx_flat.shape[0], D), jnp.float32),
        mesh=mesh,
        scratch_types=[
            pltpu.VMEM((b_per_w,), jnp.int32),
            pltpu.VMEM((CH, D), jnp.float32),
            pltpu.SemaphoreType.DMA,
        ],
        compiler_params=pltpu.CompilerParams(use_tc_tiling_on_sc=False),
    )
    def k(idx_hbm, table_hbm, out_hbm, idx_v, rows_v, sem):
        wid = lax.axis_index("s") * NC + lax.axis_index("c")
        base = wid * b_per_w
        pltpu.sync_copy(idx_hbm.at[pl.ds(base, b_per_w)], idx_v)

        def body(j, carry):
            off = j * CH
            pltpu.async_copy(
                table_hbm.at[idx_v.at[pl.ds(off, CH)]], rows_v, sem
            ).wait()
            pltpu.sync_copy(rows_v, out_hbm.at[pl.ds(base + off, CH)])
            return carry

        lax.fori_loop(0, nch, body, 0, unroll=False)

    return k(idx_flat, table)


def kernel(category_ids, embedding_table):
    batch, fields = category_ids.shape
    b = batch * fields
    idx_flat = category_ids.reshape(b).astype(jnp.int32)
    out = _sc_gather(idx_flat, embedding_table, b // NW)
    return out.reshape(batch, fields, D)


# speedup vs baseline: 1.0557x; 1.0557x over previous
import functools

import jax
import jax.numpy as jnp
from jax import lax
from jax.experimental import pallas as pl
from jax.experimental.pallas import tpu as pltpu
from jax.experimental.pallas import tpu_sc as plsc

D = 64
NC = 2
NS = 16
NW = NC * NS
CH = 512


@functools.partial(jax.jit, static_argnames=("b_per_w",))
def _sc_gather(idx_flat, table, b_per_w):
    nch = b_per_w // CH
    mesh = plsc.VectorSubcoreMesh(core_axis_name="c", subcore_axis_name="s")

    @functools.partial(
        pl.kernel,
        out_type=jax.ShapeDtypeStruct((idx_flat.shape[0], D), jnp.float32),
        mesh=mesh,
        scratch_types=[
            pltpu.VMEM((b_per_w,), jnp.int32),
            pltpu.VMEM((CH, D), jnp.float32),
            pltpu.SemaphoreType.DMA,
        ],
        compiler_params=pltpu.CompilerParams(use_tc_tiling_on_sc=False),
    )
    def k(idx_hbm, table_hbm, out_hbm, idx_v, rows_v, sem):
        wid = lax.axis_index("s") * NC + lax.axis_index("c")
        base = wid * b_per_w
        pltpu.sync_copy(idx_hbm.at[pl.ds(base, b_per_w)], idx_v)

        def body(j, carry):
            off = j * CH
            pltpu.async_copy(
                table_hbm.at[idx_v.at[pl.ds(off, CH)]], rows_v, sem
            ).wait()
            pltpu.sync_copy(rows_v, out_hbm.at[pl.ds(base + off, CH)])
            return carry

        lax.fori_loop(0, nch, body, 0, unroll=False)

    return k(idx_flat, table)


def kernel(category_ids, embedding_table):
    batch, fields = category_ids.shape
    b = batch * fields
    idx_flat = category_ids.reshape(b).astype(jnp.int32)
    out = _sc_gather(idx_flat, embedding_table, b // NW)
    return out.reshape(batch, fields, D)


# speedup vs baseline: 1.0690x; 1.0126x over previous
import functools

import jax
import jax.numpy as jnp
from jax import lax
from jax.experimental import pallas as pl
from jax.experimental.pallas import tpu as pltpu
from jax.experimental.pallas import tpu_sc as plsc

D = 64
NC = 2
NS = 16
NW = NC * NS
CH = 512
NBUF = 2


@functools.partial(jax.jit, static_argnames=("b_per_w",))
def _sc_gather(idx_flat, table, b_per_w):
    nch = b_per_w // CH
    assert nch % NBUF == 0 and nch // NBUF >= 1
    mesh = plsc.VectorSubcoreMesh(core_axis_name="c", subcore_axis_name="s")

    @functools.partial(
        pl.kernel,
        out_type=jax.ShapeDtypeStruct((idx_flat.shape[0], D), jnp.float32),
        mesh=mesh,
        scratch_types=[
            pltpu.VMEM((b_per_w,), jnp.int32),
            pltpu.VMEM((NBUF, CH, D), jnp.float32),
            [pltpu.SemaphoreType.DMA] * NBUF,
            [pltpu.SemaphoreType.DMA] * NBUF,
        ],
        compiler_params=pltpu.CompilerParams(use_tc_tiling_on_sc=False),
    )
    def k(idx_hbm, table_hbm, out_hbm, idx_v, rows_v, gsems, wsems):
        wid = lax.axis_index("s") * NC + lax.axis_index("c")
        base = wid * b_per_w
        pltpu.sync_copy(idx_hbm.at[pl.ds(base, b_per_w)], idx_v)

        def start_gather(j, p):
            pltpu.async_copy(
                table_hbm.at[idx_v.at[pl.ds(j * CH, CH)]], rows_v.at[p],
                gsems[p],
            )

        def wait_gather(p):
            pltpu.make_async_copy(
                table_hbm.at[idx_v.at[pl.ds(0, CH)]], rows_v.at[p], gsems[p]
            ).wait()

        def start_write(j, p):
            pltpu.async_copy(
                rows_v.at[p], out_hbm.at[pl.ds(base + j * CH, CH)], wsems[p]
            )

        def wait_write(p):
            pltpu.make_async_copy(
                rows_v.at[p], out_hbm.at[pl.ds(base, CH)], wsems[p]
            ).wait()

        for p in range(NBUF):
            start_gather(p, p)

        def body(g, carry):
            for p in range(NBUF):
                j = g * NBUF + p
                wait_gather(p)
                start_write(j, p)

                @pl.when(j + NBUF < nch)
                def _():
                    wait_write(p)
                    start_gather(j + NBUF, p)

            return carry

        lax.fori_loop(0, nch // NBUF, body, 0, unroll=False)
        for p in range(NBUF):
            wait_write(p)

    return k(idx_flat, table)


def kernel(category_ids, embedding_table):
    batch, fields = category_ids.shape
    b = batch * fields
    idx_flat = category_ids.reshape(b).astype(jnp.int32)
    out = _sc_gather(idx_flat, embedding_table, b // NW)
    return out.reshape(batch, fields, D)
